# baseline (speedup 1.0000x reference)
import functools

import jax
import jax.numpy as jnp
from jax import lax
from jax.experimental import pallas as pl
from jax.experimental.pallas import tpu as pltpu
from jax.experimental.pallas import tpu_sc as plsc

S = 2048
EMB = 1024
NQ = 16
NKV = 4
HD = 64
NE = 8
DFF = 2048
VOCAB_N = 32000

BM = 256
SMAX = 2 * S + NE * BM
MT = SMAX // BM
NW = 32
GCH = 64
F32 = jnp.float32
BF16 = jnp.bfloat16

_VMEM_BIG = pltpu.CompilerParams(vmem_limit_bytes=120 * 1024 * 1024)


def _gather_rows(table, idx):
    B = idx.shape[0]
    D = table.shape[1]
    bpw = B // NW
    nch = bpw // GCH
    mesh = plsc.VectorSubcoreMesh(core_axis_name="c", subcore_axis_name="s")

    @functools.partial(
        pl.kernel,
        out_type=jax.ShapeDtypeStruct((B, D), F32),
        mesh=mesh,
        scratch_types=[
            pltpu.VMEM((GCH,), jnp.int32),
            pltpu.VMEM((GCH, D), F32),
            pltpu.SemaphoreType.DMA,
        ],
    )
    def k(table_hbm, idx_hbm, out_hbm, idx_v, rows_v, sem):
        wid = lax.axis_index("s") * 2 + lax.axis_index("c")
        for c in range(nch):
            base = wid * bpw + c * GCH
            pltpu.sync_copy(idx_hbm.at[pl.ds(base, GCH)], idx_v)
            pltpu.async_copy(table_hbm.at[idx_v], rows_v, sem).wait()
            pltpu.sync_copy(rows_v, out_hbm.at[pl.ds(base, GCH)])

    return k(table, idx)


def _gather_pair(table, idx0, idx1):
    B = idx0.shape[0]
    D = table.shape[1]
    bpw = B // NW
    nch = bpw // GCH
    mesh = plsc.VectorSubcoreMesh(core_axis_name="c", subcore_axis_name="s")

    @functools.partial(
        pl.kernel,
        out_type=[
            jax.ShapeDtypeStruct((B, D), F32),
            jax.ShapeDtypeStruct((B, D), F32),
        ],
        mesh=mesh,
        scratch_types=[
            pltpu.VMEM((GCH,), jnp.int32),
            pltpu.VMEM((GCH, D), F32),
            pltpu.SemaphoreType.DMA,
        ],
    )
    def k(table_hbm, i0_hbm, i1_hbm, o0_hbm, o1_hbm, idx_v, rows_v, sem):
        wid = lax.axis_index("s") * 2 + lax.axis_index("c")
        for c in range(nch):
            base = wid * bpw + c * GCH
            pltpu.sync_copy(i0_hbm.at[pl.ds(base, GCH)], idx_v)
            pltpu.async_copy(table_hbm.at[idx_v], rows_v, sem).wait()
            pltpu.sync_copy(rows_v, o0_hbm.at[pl.ds(base, GCH)])
            pltpu.sync_copy(i1_hbm.at[pl.ds(base, GCH)], idx_v)
            pltpu.async_copy(table_hbm.at[idx_v], rows_v, sem).wait()
            pltpu.sync_copy(rows_v, o1_hbm.at[pl.ds(base, GCH)])

    return k(table, idx0, idx1)


def _halves_sum(x):
    L = x.shape[1]
    while L > 1:
        x = x[:, :L // 2] + x[:, L // 2:]
        L //= 2
    return x


def _rowmean_sq(h):
    K = h.shape[1]
    return _halves_sum(h * h) / float(K)


def _rms(h, nw_row):
    ms = _rowmean_sq(h)
    return (h * nw_row) * lax.rsqrt(ms + 1e-6)


def _res_sum(refs, nmoe):
    h = refs[0][...]
    if nmoe:
        gl = jnp.max(refs[3][...], axis=1, keepdims=True)
        gr = jnp.max(refs[4][...], axis=1, keepdims=True)
        h = h + (gl * refs[1][...] + gr * refs[2][...])
    return h


def _norm_qkv(parts, nw, wq_bf, wk_bf, wv_bf):
    nmoe = len(parts) == 5

    def body(*refs):
        n = len(parts)
        nw_ref = refs[n]
        wq_ref, wk_ref, wv_ref = refs[n + 1:n + 4]
        q_ref, k_ref, v_ref = refs[n + 4:n + 7]
        h = _res_sum(refs[:n], nmoe)
        hn = _rms(h, nw_ref[...]).astype(BF16)
        q_ref[...] = jnp.dot(hn, wq_ref[...], preferred_element_type=F32)
        k_ref[...] = jnp.dot(hn, wk_ref[...], preferred_element_type=F32)
        v_ref[...] = jnp.dot(hn, wv_ref[...], preferred_element_type=F32)

    return pl.pallas_call(
        body,
        out_shape=[
            jax.ShapeDtypeStruct((S, NQ * HD), F32),
            jax.ShapeDtypeStruct((S, NKV * HD), F32),
            jax.ShapeDtypeStruct((S, NKV * HD), F32),
        ],
        compiler_params=_VMEM_BIG,
    )(*parts, nw.reshape(1, EMB), wq_bf, wk_bf, wv_bf)


def _attention(q3, k3, v3, cs, sn):
    rep = NQ // NKV

    def body(q_ref, k_ref, v_ref, cs_ref, sn_ref, o_ref):
        cq = cs_ref[...]
        sq = sn_ref[...]
        q_ = q_ref[0]
        q1 = q_[:, :HD // 2]
        q2 = q_[:, HD // 2:]
        qr = jnp.concatenate([q1 * cq - q2 * sq, q1 * sq + q2 * cq], axis=1)
        k_ = k_ref[0]
        k1 = k_[:, :HD // 2]
        k2 = k_[:, HD // 2:]
        kr = jnp.concatenate([k1 * cq - k2 * sq, k1 * sq + k2 * cq], axis=1)
        s = lax.dot_general(qr.astype(BF16), kr.astype(BF16),
                            (((1,), (1,)), ((), ())),
                            preferred_element_type=F32) / 8.0
        rows = lax.broadcasted_iota(jnp.int32, (S, S), 0)
        cols = lax.broadcasted_iota(jnp.int32, (S, S), 1)
        s = jnp.where(cols <= rows, s, -1e9)
        mx = jnp.max(s, axis=1, keepdims=True)
        p = jnp.exp(s - mx)
        p = p / _halves_sum(p)
        o_ref[0] = jnp.dot(p.astype(BF16), v_ref[0].astype(BF16),
                           preferred_element_type=F32)

    return pl.pallas_call(
        body,
        grid=(NQ,),
        in_specs=[
            pl.BlockSpec((1, S, HD), lambda h: (h, 0, 0)),
            pl.BlockSpec((1, S, HD), lambda h: (h // rep, 0, 0)),
            pl.BlockSpec((1, S, HD), lambda h: (h // rep, 0, 0)),
            pl.BlockSpec((S, HD // 2), lambda h: (0, 0)),
            pl.BlockSpec((S, HD // 2), lambda h: (0, 0)),
        ],
        out_specs=pl.BlockSpec((1, S, HD), lambda h: (h, 0, 0)),
        out_shape=jax.ShapeDtypeStruct((NQ, S, HD), F32),
        compiler_params=_VMEM_BIG,
    )(q3, k3, v3, cs, sn)


def _wo_res_norm(parts, ao_bf, wo_bf, nw):
    nmoe = len(parts) == 5

    def body(*refs):
        ao_ref = refs[0]
        wo_ref = refs[1]
        n = len(parts)
        nw_ref = refs[2 + n]
        h2_ref = refs[3 + n]
        mi_ref = refs[4 + n]
        h = _res_sum(refs[2:2 + n], nmoe)
        h2 = h + jnp.dot(ao_ref[...], wo_ref[...], preferred_element_type=F32)
        h2_ref[...] = h2
        mi_ref[...] = _rms(h2, nw_ref[...])

    return pl.pallas_call(
        body,
        out_shape=[
            jax.ShapeDtypeStruct((S, EMB), F32),
            jax.ShapeDtypeStruct((S, EMB), F32),
        ],
        compiler_params=_VMEM_BIG,
    )(ao_bf, wo_bf, *parts, nw.reshape(1, EMB))


def _router(mi_bf, rw_bf):
    def body(mi_ref, rw_ref, dlo_ref, dhi_ref, glo_ref, ghi_ref,
             ilo_ref, ihi_ref, et_ref):
        logits = jnp.dot(mi_ref[...], rw_ref[...], preferred_element_type=F32)
        lane = lax.broadcasted_iota(jnp.int32, (S, 128), 1).astype(F32)
        logits = jnp.where(lane < NE, logits, -1e30)
        mx = jnp.max(logits, axis=1, keepdims=True)
        p = jnp.exp(logits - mx)
        p = p / jnp.sum(p, axis=1, keepdims=True)
        m1 = jnp.max(p, axis=1, keepdims=True)
        e1 = jnp.min(jnp.where(p == m1, lane, 1e9), axis=1, keepdims=True)
        oh1 = (lane == e1).astype(F32)
        p2 = jnp.where((lane == e1) | (lane >= NE), -2.0, p)
        m2 = jnp.max(p2, axis=1, keepdims=True)
        e2 = jnp.min(jnp.where(p2 == m2, lane, 1e9), axis=1, keepdims=True)
        oh2 = (lane == e2).astype(F32)
        sv = m1 + m2
        g1 = m1 / sv
        g2 = m2 / sv
        CB = 256
        li = lax.broadcasted_iota(jnp.int32, (CB, CB), 0)
        lj = lax.broadcasted_iota(jnp.int32, (CB, CB), 1)
        lower = (lj < li).astype(F32)

        def excl_cumsum(oh):
            outs = []
            carry = jnp.zeros((1, 128), F32)
            for j in range(S // CB):
                ch = lax.slice(oh, (j * CB, 0), ((j + 1) * CB, 128))
                outs.append(jnp.dot(lower, ch, preferred_element_type=F32)
                            + carry)
                carry = carry + jnp.sum(ch, axis=0, keepdims=True)
            return jnp.concatenate(outs, axis=0), carry

        cum0, tot0 = excl_cumsum(oh1)
        cum1, tot1 = excl_cumsum(oh2)
        cum1 = cum1 + tot0
        tot = tot0 + tot1
        pc = jnp.floor((tot + (BM - 1)) / BM) * BM
        ui = lax.broadcasted_iota(jnp.int32, (128, 128), 0)
        uj = lax.broadcasted_iota(jnp.int32, (128, 128), 1)
        upper = (ui < uj).astype(F32)
        pc8 = pc * jnp.ones((8, 1), F32)
        off = lax.slice(jnp.dot(pc8, upper, preferred_element_type=F32),
                        (0, 0), (1, 128))
        rank0 = jnp.sum(oh1 * cum0, axis=1, keepdims=True)
        rank1 = jnp.sum(oh2 * cum1, axis=1, keepdims=True)
        off0 = jnp.sum(oh1 * off, axis=1, keepdims=True)
        off1 = jnp.sum(oh2 * off, axis=1, keepdims=True)
        d1 = off0 + rank0
        d2 = off1 + rank1
        tok = lax.broadcasted_iota(jnp.int32, (S, 1), 0).astype(F32)
        i1 = e1 * float(S) + tok
        i2 = e2 * float(S) + tok
        ones = jnp.ones((1, 128), F32)
        lo_first = e1 < e2
        dlo_ref[...] = jnp.where(lo_first, d1, d2) * ones
        dhi_ref[...] = jnp.where(lo_first, d2, d1) * ones
        glo_ref[...] = jnp.where(lo_first, g1, g2) * ones
        ghi_ref[...] = jnp.where(lo_first, g2, g1) * ones
        ilo_ref[...] = jnp.where(lo_first, i1, i2) * ones
        ihi_ref[...] = jnp.where(lo_first, i2, i1) * ones
        tm = lax.broadcasted_iota(jnp.int32, (32, 128), 0).astype(F32) * BM
        lane2 = lax.broadcasted_iota(jnp.int32, (32, 128), 1).astype(F32)
        seg = ((tm >= off) & (tm < off + pc)).astype(F32)
        ev = jnp.sum(seg * lane2, axis=1, keepdims=True)
        valid = jnp.sum(seg, axis=1, keepdims=True) > 0.0
        et_ref[...] = jnp.where(valid, ev, -1.0) * ones

    outs = [jax.ShapeDtypeStruct((S, 128), F32) for _ in range(6)]
    outs.append(jax.ShapeDtypeStruct((32, 128), F32))
    return pl.pallas_call(body, out_shape=outs,
                          compiler_params=_VMEM_BIG)(mi_bf, rw_bf)


def _slotmap(dpair, gpair):
    BQ = 256
    gm = SMAX // BQ

    def body(dp_ref, gp_ref, src_ref, gate_ref):
        m = pl.program_id(0)
        slots = (lax.broadcasted_iota(jnp.int32, (BQ, 1), 0).astype(F32)
                 + (m * BQ).astype(F32))
        lanev = lax.broadcasted_iota(jnp.int32, (1, 128), 1).astype(F32)
        dp = dp_ref[...]
        gp = gp_ref[...]
        acc_t = jnp.zeros((BQ, 1), F32)
        acc_g = jnp.zeros((BQ, 1), F32)
        for r in range(32):
            drow = lax.slice(dp, (r, 0), (r + 1, 128))
            grow = lax.slice(gp, (r, 0), (r + 1, 128))
            eq = (drow == slots).astype(F32)
            tokrow = lanev + float((r % 16) * 128)
            acc_t = acc_t + jnp.sum(eq * tokrow, axis=1, keepdims=True)
            acc_g = acc_g + jnp.sum(eq * grow, axis=1, keepdims=True)
        ones = jnp.ones((1, 128), F32)
        src_ref[...] = acc_t * ones
        gate_ref[...] = acc_g * ones

    return pl.pallas_call(
        body,
        grid=(gm,),
        in_specs=[
            pl.BlockSpec((32, 128), lambda m: (0, 0)),
            pl.BlockSpec((32, 128), lambda m: (0, 0)),
        ],
        out_specs=[
            pl.BlockSpec((BQ, 128), lambda m: (m, 0)),
            pl.BlockSpec((BQ, 128), lambda m: (m, 0)),
        ],
        out_shape=[
            jax.ShapeDtypeStruct((SMAX, 128), F32),
            jax.ShapeDtypeStruct((SMAX, 128), F32),
        ],
    )(dpair, gpair)


def _dense_act(mi_bf, w1_bf, w3_bf):
    bn = 512
    gn = DFF // bn

    def body(m_ref, w1_ref, w3_ref, o_ref):
        mv = m_ref[...]
        u = jnp.dot(mv, w1_ref[0], preferred_element_type=F32)
        g = jnp.dot(mv, w3_ref[0], preferred_element_type=F32)
        o_ref[0] = (jax.nn.silu(u) * g).astype(BF16)

    return pl.pallas_call(
        body,
        grid=(NE, gn),
        in_specs=[
            pl.BlockSpec((S, EMB), lambda e, n: (0, 0)),
            pl.BlockSpec((1, EMB, bn), lambda e, n: (e, 0, n)),
            pl.BlockSpec((1, EMB, bn), lambda e, n: (e, 0, n)),
        ],
        out_specs=pl.BlockSpec((1, S, bn), lambda e, n: (e, 0, n)),
        out_shape=jax.ShapeDtypeStruct((NE, S, DFF), BF16),
        compiler_params=_VMEM_BIG,
    )(mi_bf, w1_bf, w3_bf)


def _dense_eo(act_bf, w2_bf):
    bn = 512
    gn = EMB // bn

    def body(a_ref, w2_ref, o_ref):
        o_ref[0] = jnp.dot(a_ref[0], w2_ref[0], preferred_element_type=F32)

    return pl.pallas_call(
        body,
        grid=(NE, gn),
        in_specs=[
            pl.BlockSpec((1, S, DFF), lambda e, n: (e, 0, 0)),
            pl.BlockSpec((1, DFF, bn), lambda e, n: (e, 0, n)),
        ],
        out_specs=pl.BlockSpec((1, S, bn), lambda e, n: (e, 0, n)),
        out_shape=jax.ShapeDtypeStruct((NE, S, EMB), F32),
        compiler_params=_VMEM_BIG,
    )(act_bf, w2_bf)


def _moe_stage1(e_tile, xs, w1_bf, w3_bf):
    bn = 512
    gn = DFF // bn

    def body(et_ref, x_ref, w1_ref, w3_ref, o_ref):
        m = pl.program_id(1)

        @pl.when(et_ref[m] >= 0)
        def _():
            x_v = x_ref[...].astype(BF16)
            u = jnp.dot(x_v, w1_ref[0], preferred_element_type=F32)
            g = jnp.dot(x_v, w3_ref[0], preferred_element_type=F32)
            o_ref[...] = (jax.nn.silu(u) * g).astype(BF16)

    def wmap(n, m, et):
        return (jnp.maximum(et[m], 0), 0, n)

    return pl.pallas_call(
        body,
        grid_spec=pltpu.PrefetchScalarGridSpec(
            num_scalar_prefetch=1,
            grid=(gn, MT),
            in_specs=[
                pl.BlockSpec((BM, EMB), lambda n, m, et: (m, 0)),
                pl.BlockSpec((1, EMB, bn), wmap),
                pl.BlockSpec((1, EMB, bn), wmap),
            ],
            out_specs=pl.BlockSpec((BM, bn), lambda n, m, et: (m, n)),
        ),
        out_shape=jax.ShapeDtypeStruct((SMAX, DFF), BF16),
    )(e_tile, xs, w1_bf, w3_bf)


def _moe_stage2(e_tile, act_bf, w2_bf):
    bn = 512
    gn = EMB // bn

    def body(et_ref, a_ref, w2_ref, o_ref):
        m = pl.program_id(1)

        @pl.when(et_ref[m] >= 0)
        def _():
            o_ref[...] = jnp.dot(a_ref[...], w2_ref[0],
                                 preferred_element_type=F32)

    def wmap(n, m, et):
        return (jnp.maximum(et[m], 0), 0, n)

    return pl.pallas_call(
        body,
        grid_spec=pltpu.PrefetchScalarGridSpec(
            num_scalar_prefetch=1,
            grid=(gn, MT),
            in_specs=[
                pl.BlockSpec((BM, DFF), lambda n, m, et: (m, 0)),
                pl.BlockSpec((1, DFF, bn), wmap),
            ],
            out_specs=pl.BlockSpec((BM, bn), lambda n, m, et: (m, n)),
        ),
        out_shape=jax.ShapeDtypeStruct((SMAX, EMB), F32),
    )(e_tile, act_bf, w2_bf)


def _final_proj(parts, nw, pw_bf, pb):
    bn = 1280
    gn = VOCAB_N // bn
    nmoe = len(parts) == 5

    def body(*refs):
        n = len(parts)
        nw_ref = refs[n]
        w_ref = refs[n + 1]
        b_ref = refs[n + 2]
        o_ref = refs[n + 3]
        h = _res_sum(refs[:n], nmoe)
        hn = _rms(h, nw_ref[...]).astype(BF16)
        o_ref[...] = (jnp.dot(hn, w_ref[...], preferred_element_type=F32)
                      + b_ref[...])

    in_specs = [pl.BlockSpec((S, pt.shape[1]), lambda nb: (0, 0))
                for pt in parts]
    in_specs += [
        pl.BlockSpec((1, EMB), lambda nb: (0, 0)),
        pl.BlockSpec((EMB, bn), lambda nb: (0, nb)),
        pl.BlockSpec((1, bn), lambda nb: (0, nb)),
    ]
    return pl.pallas_call(
        body,
        grid=(gn,),
        in_specs=in_specs,
        out_specs=pl.BlockSpec((S, bn), lambda nb: (0, nb)),
        out_shape=jax.ShapeDtypeStruct((S, VOCAB_N), F32),
        compiler_params=_VMEM_BIG,
    )(*parts, nw.reshape(1, EMB), pw_bf, pb.reshape(1, VOCAB_N))


def kernel(params, x):
    p = params
    tokens = x[0].astype(jnp.int32)
    emb = _gather_rows(p["token_emb"], tokens)

    pos = jnp.arange(S)
    inv = 1.0 / (10000.0 ** (jnp.arange(0, HD, 2, dtype=F32) / HD))
    ang = pos[:, None].astype(F32) * inv[None, :]
    cs = jnp.cos(ang)
    sn = jnp.sin(ang)

    parts = (emb,)
    for li, lp in enumerate(p["layers"]):
        wq_bf = lp["wq"].astype(BF16)
        wk_bf = lp["wk"].astype(BF16)
        wv_bf = lp["wv"].astype(BF16)
        q, k, v = _norm_qkv(parts, lp["attn_norm"], wq_bf, wk_bf, wv_bf)
        q3 = q.reshape(S, NQ, HD).transpose(1, 0, 2)
        k3 = k.reshape(S, NKV, HD).transpose(1, 0, 2)
        v3 = v.reshape(S, NKV, HD).transpose(1, 0, 2)
        ao3 = _attention(q3, k3, v3, cs, sn)
        ao_bf = ao3.transpose(1, 0, 2).reshape(S, NQ * HD).astype(BF16)
        h2, mi = _wo_res_norm(parts, ao_bf, lp["wo"].astype(BF16),
                              lp["moe_norm"])

        mi_bf = mi.astype(BF16)
        rw_bf = jnp.pad(lp["router"], ((0, 0), (0, 128 - NE))).astype(BF16)
        dlo, dhi, glo, ghi, ilo, ihi, etf = _router(mi_bf, rw_bf)

        w1_bf = lp["w1"].astype(BF16)
        w3_bf = lp["w3"].astype(BF16)
        w2_bf = lp["w2"].astype(BF16)
        if li == 0:
            act_bf = _dense_act(mi_bf, w1_bf, w3_bf)
            eo = _dense_eo(act_bf, w2_bf).reshape(NE * S, EMB)
            iL = ilo[:, 0].astype(jnp.int32)
            iH = ihi[:, 0].astype(jnp.int32)
            eoL, eoR = _gather_pair(eo, iL, iH)
        else:
            dpair = jnp.concatenate(
                [dlo[:, 0], dhi[:, 0]]).reshape(32, 128)
            gpair = jnp.concatenate(
                [glo[:, 0], ghi[:, 0]]).reshape(32, 128)
            e_tile = etf[:MT, 0].astype(jnp.int32)
            srcf, _ = _slotmap(dpair, gpair)
            src_tok = srcf[:, 0].astype(jnp.int32)
            xs = _gather_rows(mi, src_tok)
            act_bf = _moe_stage1(e_tile, xs, w1_bf, w3_bf)
            eo = _moe_stage2(e_tile, act_bf, w2_bf)
            dL = dlo[:, 0].astype(jnp.int32)
            dH = dhi[:, 0].astype(jnp.int32)
            eoL, eoR = _gather_pair(eo, dL, dH)
        parts = (h2, eoL, eoR, glo, ghi)

    logits = _final_proj(parts, p["final_norm"],
                         p["proj_w"].astype(BF16), p["proj_b"])
    return logits.reshape(1, S, VOCAB_N)
